# baseline (speedup 1.0000x reference)
import functools

import jax
import jax.numpy as jnp
from jax import lax
from jax.experimental import pallas as pl
from jax.experimental.pallas import tpu as pltpu
from jax.experimental.pallas import tpu_sc as plsc

_V = 1_000_000
_E = 64
_B = 4096
_L = 200

_NC = 2
_NS = 16
_NW = _NC * _NS
_BLK = _B // _NW

_mesh = plsc.VectorSubcoreMesh(
    core_axis_name="c", subcore_axis_name="s", num_cores=_NC, num_subcores=_NS
)


@functools.partial(
    pl.kernel,
    mesh=_mesh,
    out_type=jax.ShapeDtypeStruct((_B, _L, _E), jnp.float32),
    scratch_types=[
        pltpu.VMEM((_BLK, _L), jnp.int32),
        pltpu.VMEM((_L, _E), jnp.float32),
        pltpu.VMEM((_L, _E), jnp.float32),
        pltpu.SemaphoreType.DMA,
        pltpu.SemaphoreType.DMA,
        pltpu.SemaphoreType.DMA,
        pltpu.SemaphoreType.DMA,
    ],
    compiler_params=pltpu.CompilerParams(use_tc_tiling_on_sc=False),
)
def _gather_kernel(x_hbm, table_hbm, out_hbm, idx_v, rows_a, rows_b,
                   sg_a, sg_b, sw_a, sw_b):
    wid = lax.axis_index("s") * _NC + lax.axis_index("c")
    b0 = wid * _BLK

    pltpu.sync_copy(x_hbm.at[pl.ds(b0, _BLK)], idx_v)

    def fire_gather(i, buf, sem):
        pltpu.async_copy(table_hbm.at[idx_v.at[i]], buf, sem)

    def wait_gather(buf, sem):
        pltpu.make_async_copy(table_hbm.at[pl.ds(0, _L)], buf, sem).wait()

    def fire_wb(i, buf, sem):
        pltpu.async_copy(buf, out_hbm.at[b0 + i], sem)

    def wait_wb(buf, sem):
        pltpu.make_async_copy(buf, out_hbm.at[0], sem).wait()

    fire_gather(0, rows_a, sg_a)

    @pl.loop(0, _BLK // 2)
    def _pair(p):
        i0 = 2 * p
        wait_gather(rows_a, sg_a)

        @pl.when(p > 0)
        def _():
            wait_wb(rows_b, sw_b)

        fire_gather(i0 + 1, rows_b, sg_b)
        fire_wb(i0, rows_a, sw_a)
        wait_gather(rows_b, sg_b)

        @pl.when(p + 1 < _BLK // 2)
        def _():
            wait_wb(rows_a, sw_a)
            fire_gather(i0 + 2, rows_a, sg_a)

        fire_wb(i0 + 1, rows_b, sw_b)

    wait_wb(rows_a, sw_a)
    wait_wb(rows_b, sw_b)


def kernel(x, vocab):
    table = vocab.reshape(_V, _E)
    return _gather_kernel(x, table)


# speedup vs baseline: 1.0630x; 1.0630x over previous
import functools

import jax
import jax.numpy as jnp
from jax import lax
from jax.experimental import pallas as pl
from jax.experimental.pallas import tpu as pltpu
from jax.experimental.pallas import tpu_sc as plsc

_V = 1_000_000
_E = 64
_B = 4096
_L = 200

_NC = 2
_NS = 16
_NW = _NC * _NS
_BLK = _B // _NW
_K = 4
_NCHUNK = _L // _K

_mesh = plsc.VectorSubcoreMesh(
    core_axis_name="c", subcore_axis_name="s", num_cores=_NC, num_subcores=_NS
)


@functools.partial(
    pl.kernel,
    mesh=_mesh,
    out_type=jax.ShapeDtypeStruct((_L, _B, _E), jnp.float32),
    scratch_types=[
        pltpu.VMEM((_L, _BLK), jnp.int32),
        pltpu.VMEM((_K, _BLK, _E), jnp.float32),
        pltpu.VMEM((_K, _BLK, _E), jnp.float32),
        pltpu.SemaphoreType.DMA,
        pltpu.SemaphoreType.DMA,
        pltpu.SemaphoreType.DMA,
        pltpu.SemaphoreType.DMA,
    ],
    compiler_params=pltpu.CompilerParams(use_tc_tiling_on_sc=False),
)
def _gather_kernel(xt_hbm, table_hbm, out_hbm, idx_v, rows_a, rows_b,
                   sg_a, sg_b, sw_a, sw_b):
    wid = lax.axis_index("s") * _NC + lax.axis_index("c")
    b0 = wid * _BLK

    pltpu.sync_copy(xt_hbm.at[:, pl.ds(b0, _BLK)], idx_v)

    def fire_gather(chunk, buf, sem):
        return [
            pltpu.async_copy(
                table_hbm.at[idx_v.at[chunk * _K + j]], buf.at[j], sem
            )
            for j in range(_K)
        ]

    def fire_wb(chunk, buf, sem):
        return pltpu.async_copy(
            buf, out_hbm.at[pl.ds(chunk * _K, _K), pl.ds(b0, _BLK)], sem
        )

    def wait_wb(buf, sem):
        pltpu.make_async_copy(
            buf, out_hbm.at[pl.ds(0, _K), pl.ds(b0, _BLK)], sem
        ).wait()

    @pl.loop(0, _NCHUNK // 2)
    def _pair(p):
        c0 = 2 * p

        @pl.when(p > 0)
        def _():
            wait_wb(rows_a, sw_a)

        g0 = fire_gather(c0, rows_a, sg_a)

        @pl.when(p > 0)
        def _():
            wait_wb(rows_b, sw_b)

        g1 = fire_gather(c0 + 1, rows_b, sg_b)
        for c in g0:
            c.wait()
        fire_wb(c0, rows_a, sw_a)
        for c in g1:
            c.wait()
        fire_wb(c0 + 1, rows_b, sw_b)

    wait_wb(rows_a, sw_a)
    wait_wb(rows_b, sw_b)


def kernel(x, vocab):
    table = vocab.reshape(_V, _E)
    out_lbe = _gather_kernel(x.T, table)
    return jnp.transpose(out_lbe, (1, 0, 2))
